# Initial kernel scaffold; baseline (speedup 1.0000x reference)
import jax
import jax.numpy as jnp
from jax.experimental import pallas as pl


def kernel(position, units_mask, relic_positions, relic_mask, tile_type, normalized_reward, tile_table, unit_table):
    raise NotImplementedError("write your pallas kernel here")


# speedup vs baseline: 6.7936x; 6.7936x over previous
import functools

import jax
import jax.numpy as jnp
from jax import lax
from jax.experimental import pallas as pl
from jax.experimental.pallas import tpu as pltpu
from jax.experimental.pallas import tpu_sc as plsc

_B, _M, _T, _U, _W, _H = 64, 32, 2, 16, 24, 24
_C = 24
_E = _B * _M
_CELLS = _W * _H
_MAPF = _CELLS * _C
_NW = 32
_EPW = _E // _NW


def _sc_body(ints_hbm, flts_hbm, ttab_hbm, utab_hbm, out_hbm,
             ints_v, flts_v, ttab_v, utab_v, map_v):
    wid = lax.axis_index("c") * 16 + lax.axis_index("s")

    pltpu.sync_copy(ttab_hbm, ttab_v)
    pltpu.sync_copy(utab_hbm, utab_v)

    lanes = lax.iota(jnp.int32, 16)
    iota24 = lanes * _C
    zero_v = jnp.zeros((16,), jnp.float32)
    zero_i = jnp.zeros((16,), jnp.int32)
    mask9 = lanes < 9
    mask1 = lanes < 1
    choff = [
        jnp.where(lanes == 0, 4 + t,
                  jnp.where(lanes <= 8, 5 + 8 * t + lanes, 0)).astype(jnp.int32)
        for t in range(_T)
    ]

    def env_body(i, carry):
        env = wid * _EPW + i
        pltpu.sync_copy(ints_hbm.at[env], ints_v)
        pltpu.sync_copy(flts_hbm.at[env], flts_v)

        def zbody(j, c):
            base = j * 256
            for k in range(16):
                map_v[pl.ds(base + k * 16, 16)] = zero_v
            return c
        lax.fori_loop(0, _MAPF // 256, zbody, 0)

        nr_vec = flts_v[pl.ds(48, 16)]

        def gbody(g, c):
            tt = ints_v[pl.ds(g * 16, 16)]
            idx4 = tt * 4
            addr = iota24 + g * (16 * _C)
            for e in range(4):
                vals = plsc.load_gather(ttab_v, [idx4 + e])
                plsc.store_scatter(map_v, [addr + e], vals)
            plsc.store_scatter(map_v, [addr + 23], nr_vec)
            return c
        lax.fori_loop(0, _CELLS // 16, gbody, 0)

        for t in range(_T):
            xv = ints_v[pl.ds(576 + 32 * t, 16)]
            yv = ints_v[pl.ds(592 + 32 * t, 16)]
            av = (xv * _H + yv) * _C
            mv = flts_v[pl.ds(16 * t, 16)]
            for u in range(_U):
                plsc.addupdate_scatter(map_v, [choff[t] + av[u]],
                                       mv[u] * utab_v[u], mask=mask9)

        rxv = ints_v[pl.ds(640, 16)]
        ryv = ints_v[pl.ds(656, 16)]
        rav = (rxv * _H + ryv) * _C + 22
        rmv = flts_v[pl.ds(32, 16)]
        for r in range(6):
            plsc.addupdate_scatter(map_v, [zero_i + rav[r]],
                                   zero_v + rmv[r], mask=mask1)

        pltpu.sync_copy(map_v, out_hbm.at[env])
        return carry

    lax.fori_loop(0, _EPW, env_body, 0)


def _sc_call(ints, flts, ttab, utab):
    mesh = plsc.VectorSubcoreMesh(core_axis_name="c", subcore_axis_name="s")
    return pl.kernel(
        _sc_body,
        out_type=jax.ShapeDtypeStruct((_E, _MAPF), jnp.float32),
        mesh=mesh,
        compiler_params=pltpu.CompilerParams(needs_layout_passes=False),
        scratch_types=[
            pltpu.VMEM((672,), jnp.int32),
            pltpu.VMEM((64,), jnp.float32),
            pltpu.VMEM((16,), jnp.float32),
            pltpu.VMEM((16, 16), jnp.float32),
            pltpu.VMEM((_MAPF,), jnp.float32),
        ],
    )(ints, flts, ttab, utab)


def kernel(position, units_mask, relic_positions, relic_mask, tile_type,
           normalized_reward, tile_table, unit_table):
    f32 = jnp.float32
    tt = tile_type.reshape(_E, _CELLS).astype(jnp.int32)
    pos = position.reshape(_E, _T, _U, 2)
    x = pos[..., 0]
    y = pos[..., 1]
    rp = relic_positions.reshape(_E, 6, 2)
    padi = jnp.zeros((_E, 10), jnp.int32)
    ints = jnp.concatenate(
        [tt, x[:, 0], y[:, 0], x[:, 1], y[:, 1],
         jnp.concatenate([rp[..., 0], padi], 1),
         jnp.concatenate([rp[..., 1], padi], 1)], axis=1)
    um = units_mask.reshape(_E, _T, _U).astype(f32)
    rm = jnp.concatenate([relic_mask.reshape(_E, 6).astype(f32),
                          jnp.zeros((_E, 10), f32)], axis=1)
    nrv = jnp.broadcast_to(normalized_reward.reshape(_E, 1).astype(f32),
                           (_E, 16))
    flts = jnp.concatenate([um[:, 0], um[:, 1], rm, nrv], axis=1)
    ttab = jnp.concatenate([tile_table.reshape(12).astype(f32),
                            jnp.zeros((4,), f32)])
    utab = jnp.concatenate(
        [jnp.full((_U, 1), 1.0 / _U, f32), unit_table.astype(f32),
         jnp.zeros((_U, 7), f32)], axis=1)
    out = _sc_call(ints, flts, ttab, utab)
    return out.reshape(_B, _M, _W, _H, _C)


# speedup vs baseline: 7.2243x; 1.0634x over previous
import functools

import jax
import jax.numpy as jnp
from jax import lax
from jax.experimental import pallas as pl
from jax.experimental.pallas import tpu as pltpu
from jax.experimental.pallas import tpu_sc as plsc

_B, _M, _T, _U, _W, _H = 64, 32, 2, 16, 24, 24
_C = 24
_E = _B * _M
_CELLS = _W * _H
_NW = 32
_EPW = _E // _NW


def _sc_body(ints_hbm, flts_hbm, ttab_hbm, utab_hbm, out_hbm,
             ints_v, flts_v, ttab_v, utab_v, map_v):
    wid = lax.axis_index("c") * 16 + lax.axis_index("s")
    out2 = out_hbm.reshape(_E, _CELLS, _C)

    pltpu.sync_copy(ttab_hbm, ttab_v)
    pltpu.sync_copy(utab_hbm, utab_v)

    lanes = lax.iota(jnp.int32, 16)
    zero_v = jnp.zeros((16,), jnp.float32)
    zero_i = jnp.zeros((16,), jnp.int32)
    mask9 = lanes < 9
    mask1 = lanes < 1
    choff = [
        jnp.where(lanes == 0, 4 + t,
                  jnp.where(lanes <= 8, 5 + 8 * t + lanes, 0)).astype(jnp.int32)
        for t in range(_T)
    ]

    def env_body(i, carry):
        env = wid * _EPW + i
        pltpu.sync_copy(ints_hbm.at[env], ints_v)
        pltpu.sync_copy(flts_hbm.at[env], flts_v)

        def zbody(j, c):
            for k in range(4):
                map_v[j * 4 + k, pl.ds(0, 16)] = zero_v
                map_v[j * 4 + k, pl.ds(8, 16)] = zero_v
            return c
        lax.fori_loop(0, _CELLS // 4, zbody, 0)

        nr_vec = flts_v[pl.ds(48, 16)]

        def gbody(g, c):
            tt = ints_v[pl.ds(g * 16, 16)]
            idx4 = tt * 4
            cells = lanes + g * 16
            for e in range(4):
                vals = plsc.load_gather(ttab_v, [idx4 + e])
                plsc.store_scatter(map_v, [cells, zero_i + e], vals)
            plsc.store_scatter(map_v, [cells, zero_i + 23], nr_vec)
            return c
        lax.fori_loop(0, _CELLS // 16, gbody, 0)

        for t in range(_T):
            xv = ints_v[pl.ds(576 + 32 * t, 16)]
            yv = ints_v[pl.ds(592 + 32 * t, 16)]
            cv = xv * _H + yv
            mv = flts_v[pl.ds(16 * t, 16)]
            for u in range(_U):
                plsc.addupdate_scatter(map_v, [zero_i + cv[u], choff[t]],
                                       mv[u] * utab_v[u], mask=mask9)

        rxv = ints_v[pl.ds(640, 16)]
        ryv = ints_v[pl.ds(656, 16)]
        rcv = rxv * _H + ryv
        rmv = flts_v[pl.ds(32, 16)]
        for r in range(6):
            plsc.addupdate_scatter(map_v, [zero_i + rcv[r], zero_i + 22],
                                   zero_v + rmv[r], mask=mask1)

        pltpu.sync_copy(map_v, out2.at[env])
        return carry

    lax.fori_loop(0, _EPW, env_body, 0)


def _sc_call(ints, flts, ttab, utab):
    mesh = plsc.VectorSubcoreMesh(core_axis_name="c", subcore_axis_name="s")
    return pl.kernel(
        _sc_body,
        out_type=jax.ShapeDtypeStruct((_B, _M, _W, _H, _C), jnp.float32),
        mesh=mesh,
        compiler_params=pltpu.CompilerParams(needs_layout_passes=False),
        scratch_types=[
            pltpu.VMEM((672,), jnp.int32),
            pltpu.VMEM((64,), jnp.float32),
            pltpu.VMEM((16,), jnp.float32),
            pltpu.VMEM((16, 16), jnp.float32),
            pltpu.VMEM((_CELLS, _C), jnp.float32),
        ],
    )(ints, flts, ttab, utab)


def kernel(position, units_mask, relic_positions, relic_mask, tile_type,
           normalized_reward, tile_table, unit_table):
    f32 = jnp.float32
    tt = tile_type.reshape(_E, _CELLS).astype(jnp.int32)
    pos = position.reshape(_E, _T, _U, 2)
    x = pos[..., 0]
    y = pos[..., 1]
    rp = relic_positions.reshape(_E, 6, 2)
    padi = jnp.zeros((_E, 10), jnp.int32)
    ints = jnp.concatenate(
        [tt, x[:, 0], y[:, 0], x[:, 1], y[:, 1],
         jnp.concatenate([rp[..., 0], padi], 1),
         jnp.concatenate([rp[..., 1], padi], 1)], axis=1)
    um = units_mask.reshape(_E, _T, _U).astype(f32)
    rm = jnp.concatenate([relic_mask.reshape(_E, 6).astype(f32),
                          jnp.zeros((_E, 10), f32)], axis=1)
    nrv = jnp.broadcast_to(normalized_reward.reshape(_E, 1).astype(f32),
                           (_E, 16))
    flts = jnp.concatenate([um[:, 0], um[:, 1], rm, nrv], axis=1)
    ttab = jnp.concatenate([tile_table.reshape(12).astype(f32),
                            jnp.zeros((4,), f32)])
    utab = jnp.concatenate(
        [jnp.full((_U, 1), 1.0 / _U, f32), unit_table.astype(f32),
         jnp.zeros((_U, 7), f32)], axis=1)
    return _sc_call(ints, flts, ttab, utab)


# speedup vs baseline: 12.6883x; 1.7563x over previous
import functools

import jax
import jax.numpy as jnp
from jax import lax
from jax.experimental import pallas as pl
from jax.experimental.pallas import tpu as pltpu
from jax.experimental.pallas import tpu_sc as plsc

_B, _M, _T, _U, _W, _H = 64, 32, 2, 16, 24, 24
_C = 24
_CELLS = _W * _H
_NB = 4
_NCH = 24
_WCH = _W // _NCH
_CCEL = _WCH * _H
_CROWS = _CCEL * _C


def _sc_body(tt_hbm, ints_hbm, flts_hbm, ttab_hbm, utab_hbm, out_hbm,
             tt_v, ints_v, flts_v, ttab_v, utab_v, buf_v):
    m = lax.axis_index("c") * 16 + lax.axis_index("s")
    out3 = out_hbm.reshape(_M, _CELLS * _C, _B)

    pltpu.sync_copy(ttab_hbm, ttab_v)
    pltpu.sync_copy(utab_hbm, utab_v)

    lanes = lax.iota(jnp.int32, 16)
    zero_v = jnp.zeros((16,), jnp.float32)

    def chunk_body(ci, carry0):
        w0 = ci * _WCH

        def zero_body(lc, carry1):
            row0 = lc * _C
            for c in range(4, 23):
                for cb in range(_NB):
                    buf_v[row0 + c, pl.ds(cb * 16, 16)] = zero_v
            return carry1
        lax.fori_loop(0, _CCEL, zero_body, 0)

        def bblk_body(bb, carry1):
            boff = bb * 16
            pltpu.sync_copy(tt_hbm.at[m, bb, pl.ds(ci * _CCEL, _CCEL)], tt_v)
            pltpu.sync_copy(ints_hbm.at[m, bb], ints_v)
            pltpu.sync_copy(flts_hbm.at[m, bb], flts_v)
            nrv = flts_v[38]

            def cell_body(lc, carry2):
                row0 = lc * _C
                idx4 = tt_v[lc] * 4
                for e in range(4):
                    buf_v[row0 + e, pl.ds(boff, 16)] = (
                        plsc.load_gather(ttab_v, [idx4 + e]))
                buf_v[row0 + 23, pl.ds(boff, 16)] = nrv
                return carry2
            lax.fori_loop(0, _CCEL, cell_body, 0)

            for t in range(_T):
                for u in range(_U):
                    xv = ints_v[t * 16 + u]
                    yv = ints_v[32 + t * 16 + u]
                    mv = flts_v[t * 16 + u]
                    inb = (xv >= w0) & (xv < w0 + _WCH)
                    meff = jnp.where(inb, mv, 0.0)
                    rowv = ((xv - w0) * _H + yv) * _C
                    urow = utab_v[u]
                    for e in range(9):
                        c = 4 + t if e == 0 else 5 + 8 * t + e
                        plsc.addupdate_scatter(
                            buf_v, [rowv + c, boff + lanes],
                            meff * urow[e], mask=inb)

            for r in range(6):
                xv = ints_v[64 + r]
                yv = ints_v[70 + r]
                mv = flts_v[32 + r]
                inb = (xv >= w0) & (xv < w0 + _WCH)
                meff = jnp.where(inb, mv, 0.0)
                rowv = ((xv - w0) * _H + yv) * _C + 22
                plsc.addupdate_scatter(buf_v, [rowv, boff + lanes],
                                       meff, mask=inb)
            return carry1
        lax.fori_loop(0, _NB, bblk_body, 0)

        pltpu.sync_copy(buf_v, out3.at[m, pl.ds(ci * _CROWS, _CROWS), :])
        return carry0

    lax.fori_loop(0, _NCH, chunk_body, 0)


def _sc_call(tt, ints, flts, ttab, utab):
    mesh = plsc.VectorSubcoreMesh(core_axis_name="c", subcore_axis_name="s")
    return pl.kernel(
        _sc_body,
        out_type=jax.ShapeDtypeStruct((_M, _W, _H, _C, _B), jnp.float32),
        mesh=mesh,
        compiler_params=pltpu.CompilerParams(needs_layout_passes=False),
        scratch_types=[
            pltpu.VMEM((_CCEL, 16), jnp.int32),
            pltpu.VMEM((76, 16), jnp.int32),
            pltpu.VMEM((40, 16), jnp.float32),
            pltpu.VMEM((16,), jnp.float32),
            pltpu.VMEM((16, 16), jnp.float32),
            pltpu.VMEM((_CROWS, _B), jnp.float32),
        ],
    )(tt, ints, flts, ttab, utab)


def kernel(position, units_mask, relic_positions, relic_mask, tile_type,
           normalized_reward, tile_table, unit_table):
    f32 = jnp.float32
    i32 = jnp.int32
    tt = (tile_type.astype(i32).transpose(1, 2, 3, 0)
          .reshape(_M, _CELLS, _NB, 16).transpose(0, 2, 1, 3))
    x = position[..., 0].transpose(1, 2, 3, 0).reshape(_M, _T * _U, _B)
    y = position[..., 1].transpose(1, 2, 3, 0).reshape(_M, _T * _U, _B)
    rx = relic_positions[..., 0].transpose(1, 2, 0)
    ry = relic_positions[..., 1].transpose(1, 2, 0)
    ints = (jnp.concatenate([x, y, rx, ry], axis=1)
            .reshape(_M, 76, _NB, 16).transpose(0, 2, 1, 3))
    um = units_mask.astype(f32).transpose(1, 2, 3, 0).reshape(_M, _T * _U, _B)
    rm = relic_mask.astype(f32).transpose(1, 2, 0)
    nr = normalized_reward.astype(f32).T[:, None, :]
    flts = (jnp.concatenate([um, rm, nr, jnp.zeros((_M, 1, _B), f32)], axis=1)
            .reshape(_M, 40, _NB, 16).transpose(0, 2, 1, 3))
    ttab = jnp.concatenate([tile_table.reshape(12).astype(f32),
                            jnp.zeros((4,), f32)])
    utab = jnp.concatenate(
        [jnp.full((_U, 1), 1.0 / _U, f32), unit_table.astype(f32),
         jnp.zeros((_U, 7), f32)], axis=1)
    out = _sc_call(tt, ints, flts, ttab, utab)
    return out.transpose(4, 0, 1, 2, 3)


# speedup vs baseline: 37.5524x; 2.9596x over previous
import functools

import jax
import jax.numpy as jnp
from jax import lax
from jax.experimental import pallas as pl
from jax.experimental.pallas import tpu as pltpu
from jax.experimental.pallas import tpu_sc as plsc

_B, _M, _T, _U, _W, _H = 64, 32, 2, 16, 24, 24
_C = 24
_CELLS = _W * _H
_NB = 4
_CCEL = _H
_CROWS = _CCEL * _C
_HROWS = _CROWS // 2


def _sc_body(tt_hbm, ints_hbm, flts_hbm, ttab_hbm, utab_hbm, out_hbm,
             tt_v, ints_v, flts_v, ttab_v, utab_v, buf_v,
             ttsem, osemA, osemB):
    m = lax.axis_index("c") * 16 + lax.axis_index("s")
    out3 = out_hbm.reshape(_M, _CELLS * _C, _B)

    pltpu.sync_copy(ttab_hbm, ttab_v)
    pltpu.sync_copy(utab_hbm, utab_v)
    pltpu.sync_copy(ints_hbm.at[m], ints_v)
    pltpu.sync_copy(flts_hbm.at[m], flts_v)
    pltpu.async_copy(tt_hbm.at[m, pl.ds(0, _CCEL), :], tt_v.at[0], ttsem)

    lanes = lax.iota(jnp.int32, 16)
    zero_v = jnp.zeros((16,), jnp.float32)

    def half_out(ci, h):
        return out3.at[m, pl.ds(ci * _CROWS + h * _HROWS, _HROWS), :]

    def chunk_body(ci, carry0):
        par = lax.rem(ci, 2)
        w0 = ci

        pltpu.make_async_copy(tt_hbm.at[m, pl.ds(0, _CCEL), :],
                              tt_v.at[0], ttsem).wait()

        @pl.when(ci < _W - 1)
        def _():
            pltpu.async_copy(tt_hbm.at[m, pl.ds((ci + 1) * _CCEL, _CCEL), :],
                             tt_v.at[1 - par], ttsem)

        @pl.when(ci > 0)
        def _():
            pltpu.make_async_copy(buf_v.at[pl.ds(0, _HROWS)],
                                  half_out(ci, 0), osemA).wait()
            pltpu.make_async_copy(buf_v.at[pl.ds(_HROWS, _HROWS)],
                                  half_out(ci, 1), osemB).wait()

        def zero_body(lc, carry1):
            row0 = lc * _C
            for c in range(4, 23):
                for cb in range(_NB):
                    buf_v[row0 + c, pl.ds(cb * 16, 16)] = zero_v
            return carry1
        lax.fori_loop(0, _CCEL, zero_body, 0)

        def bblk_scatter(bb, carry1):
            boff = bb * 16
            for t in range(_T):
                for u in range(_U):
                    xv = ints_v[t * 16 + u, pl.ds(boff, 16)]
                    yv = ints_v[32 + t * 16 + u, pl.ds(boff, 16)]
                    mv = flts_v[t * 16 + u, pl.ds(boff, 16)]
                    inb = xv == w0
                    meff = jnp.where(inb, mv, 0.0)
                    rowv = yv * _C
                    urow = utab_v[u]
                    for e in range(9):
                        c = 4 + t if e == 0 else 5 + 8 * t + e
                        plsc.addupdate_scatter(
                            buf_v, [rowv + c, boff + lanes],
                            meff * urow[e], mask=inb)
            for r in range(6):
                xv = ints_v[64 + r, pl.ds(boff, 16)]
                yv = ints_v[70 + r, pl.ds(boff, 16)]
                mv = flts_v[32 + r, pl.ds(boff, 16)]
                inb = xv == w0
                meff = jnp.where(inb, mv, 0.0)
                rowv = yv * _C + 22
                plsc.addupdate_scatter(buf_v, [rowv, boff + lanes],
                                       meff, mask=inb)
            return carry1
        lax.fori_loop(0, _NB, bblk_scatter, 0)

        def dense(lc0, carry1):
            def bblk_dense(bb, carry2):
                boff = bb * 16
                nrv = flts_v[38, pl.ds(boff, 16)]

                def cell_body(lc, carry3):
                    row0 = lc * _C
                    idx4 = tt_v[par, lc, pl.ds(boff, 16)] * 4
                    for e in range(4):
                        buf_v[row0 + e, pl.ds(boff, 16)] = (
                            plsc.load_gather(ttab_v, [idx4 + e]))
                    buf_v[row0 + 23, pl.ds(boff, 16)] = nrv
                    return carry3
                lax.fori_loop(lc0, lc0 + _CCEL // 2, cell_body, 0)
                return carry2
            return lax.fori_loop(0, _NB, bblk_dense, carry1)

        dense(0, 0)
        pltpu.async_copy(buf_v.at[pl.ds(0, _HROWS)], half_out(ci, 0), osemA)
        dense(_CCEL // 2, 0)
        pltpu.async_copy(buf_v.at[pl.ds(_HROWS, _HROWS)], half_out(ci, 1),
                         osemB)
        return carry0

    lax.fori_loop(0, _W, chunk_body, 0)
    pltpu.make_async_copy(buf_v.at[pl.ds(0, _HROWS)],
                          half_out(0, 0), osemA).wait()
    pltpu.make_async_copy(buf_v.at[pl.ds(_HROWS, _HROWS)],
                          half_out(0, 1), osemB).wait()


def _sc_call(tt, ints, flts, ttab, utab):
    mesh = plsc.VectorSubcoreMesh(core_axis_name="c", subcore_axis_name="s")
    return pl.kernel(
        _sc_body,
        out_type=jax.ShapeDtypeStruct((_M, _W, _H, _C, _B), jnp.float32),
        mesh=mesh,
        compiler_params=pltpu.CompilerParams(needs_layout_passes=False),
        scratch_types=[
            pltpu.VMEM((2, _CCEL, _B), jnp.int32),
            pltpu.VMEM((76, _B), jnp.int32),
            pltpu.VMEM((40, _B), jnp.float32),
            pltpu.VMEM((16,), jnp.float32),
            pltpu.VMEM((16, 16), jnp.float32),
            pltpu.VMEM((_CROWS, _B), jnp.float32),
            pltpu.SemaphoreType.DMA,
            pltpu.SemaphoreType.DMA,
            pltpu.SemaphoreType.DMA,
        ],
    )(tt, ints, flts, ttab, utab)


def kernel(position, units_mask, relic_positions, relic_mask, tile_type,
           normalized_reward, tile_table, unit_table):
    f32 = jnp.float32
    i32 = jnp.int32
    tt = tile_type.astype(i32).transpose(1, 2, 3, 0).reshape(_M, _CELLS, _B)
    x = position[..., 0].transpose(1, 2, 3, 0).reshape(_M, _T * _U, _B)
    y = position[..., 1].transpose(1, 2, 3, 0).reshape(_M, _T * _U, _B)
    rx = relic_positions[..., 0].transpose(1, 2, 0)
    ry = relic_positions[..., 1].transpose(1, 2, 0)
    ints = jnp.concatenate([x, y, rx, ry], axis=1)
    um = units_mask.astype(f32).transpose(1, 2, 3, 0).reshape(_M, _T * _U, _B)
    rm = relic_mask.astype(f32).transpose(1, 2, 0)
    nr = normalized_reward.astype(f32).T[:, None, :]
    flts = jnp.concatenate([um, rm, nr, jnp.zeros((_M, 1, _B), f32)], axis=1)
    ttab = jnp.concatenate([tile_table.reshape(12).astype(f32),
                            jnp.zeros((4,), f32)])
    utab = jnp.concatenate(
        [jnp.full((_U, 1), 1.0 / _U, f32), unit_table.astype(f32),
         jnp.zeros((_U, 7), f32)], axis=1)
    out = _sc_call(tt, ints, flts, ttab, utab)
    return out.transpose(4, 0, 1, 2, 3)


# speedup vs baseline: 37.5840x; 1.0008x over previous
import functools

import jax
import jax.numpy as jnp
from jax import lax
from jax.experimental import pallas as pl
from jax.experimental.pallas import tpu as pltpu
from jax.experimental.pallas import tpu_sc as plsc

_B, _M, _T, _U, _W, _H = 64, 32, 2, 16, 24, 24
_C = 24
_CELLS = _W * _H
_NB = 4
_CCEL = _H
_CROWS = _CCEL * _C
_HROWS = _CROWS // 2


def _sc_body(tt_hbm, ints_hbm, flts_hbm, ttab_hbm, utab_hbm, out_hbm,
             tt_v, ints_v, flts_v, ttab_v, utab_v, buf_v,
             ttsem, osemA, osemB):
    m = lax.axis_index("c") * 16 + lax.axis_index("s")
    out3 = out_hbm.reshape(_M, _CELLS * _C, _B)

    pltpu.sync_copy(ttab_hbm, ttab_v)
    pltpu.sync_copy(utab_hbm, utab_v)
    pltpu.sync_copy(ints_hbm.at[m], ints_v)
    pltpu.sync_copy(flts_hbm.at[m], flts_v)
    pltpu.async_copy(tt_hbm.at[m, pl.ds(0, _CCEL), :], tt_v.at[0], ttsem)

    lanes = lax.iota(jnp.int32, 16)
    zero_v = jnp.zeros((16,), jnp.float32)

    def half_out(ci, h):
        return out3.at[m, pl.ds(ci * _CROWS + h * _HROWS, _HROWS), :]

    def chunk_body(ci, carry0):
        par = lax.rem(ci, 2)
        w0 = ci

        with jax.named_scope("waits"):
            pltpu.make_async_copy(tt_hbm.at[m, pl.ds(0, _CCEL), :],
                                  tt_v.at[0], ttsem).wait()

            @pl.when(ci < _W - 1)
            def _():
                pltpu.async_copy(
                    tt_hbm.at[m, pl.ds((ci + 1) * _CCEL, _CCEL), :],
                    tt_v.at[1 - par], ttsem)

            @pl.when(ci > 0)
            def _():
                pltpu.make_async_copy(buf_v.at[pl.ds(0, _HROWS)],
                                      half_out(ci, 0), osemA).wait()
                pltpu.make_async_copy(buf_v.at[pl.ds(_HROWS, _HROWS)],
                                      half_out(ci, 1), osemB).wait()

        def zero_body(lc, carry1):
            row0 = lc * _C
            for c in range(4, 23):
                for cb in range(_NB):
                    buf_v[row0 + c, pl.ds(cb * 16, 16)] = zero_v
            return carry1
        with jax.named_scope("zero"):
            lax.fori_loop(0, _CCEL, zero_body, 0)

        def bblk_scatter(bb, carry1):
            boff = bb * 16
            for t in range(_T):
                for u in range(_U):
                    xv = ints_v[t * 16 + u, pl.ds(boff, 16)]
                    yv = ints_v[32 + t * 16 + u, pl.ds(boff, 16)]
                    mv = flts_v[t * 16 + u, pl.ds(boff, 16)]
                    inb = xv == w0
                    meff = jnp.where(inb, mv, 0.0)
                    rowv = yv * _C
                    urow = utab_v[u]
                    for e in range(9):
                        c = 4 + t if e == 0 else 5 + 8 * t + e
                        plsc.addupdate_scatter(
                            buf_v, [rowv + c, boff + lanes],
                            meff * urow[e], mask=inb)
            for r in range(6):
                xv = ints_v[64 + r, pl.ds(boff, 16)]
                yv = ints_v[70 + r, pl.ds(boff, 16)]
                mv = flts_v[32 + r, pl.ds(boff, 16)]
                inb = xv == w0
                meff = jnp.where(inb, mv, 0.0)
                rowv = yv * _C + 22
                plsc.addupdate_scatter(buf_v, [rowv, boff + lanes],
                                       meff, mask=inb)
            return carry1
        with jax.named_scope("scatter"):
            lax.fori_loop(0, _NB, bblk_scatter, 0)

        def dense(lc0, carry1):
            def bblk_dense(bb, carry2):
                boff = bb * 16
                nrv = flts_v[38, pl.ds(boff, 16)]

                def cell_body(lc, carry3):
                    row0 = lc * _C
                    idx4 = tt_v[par, lc, pl.ds(boff, 16)] * 4
                    for e in range(4):
                        buf_v[row0 + e, pl.ds(boff, 16)] = (
                            plsc.load_gather(ttab_v, [idx4 + e]))
                    buf_v[row0 + 23, pl.ds(boff, 16)] = nrv
                    return carry3
                lax.fori_loop(lc0, lc0 + _CCEL // 2, cell_body, 0)
                return carry2
            return lax.fori_loop(0, _NB, bblk_dense, carry1)

        with jax.named_scope("dense"):
            dense(0, 0)
            pltpu.async_copy(buf_v.at[pl.ds(0, _HROWS)], half_out(ci, 0),
                             osemA)
            dense(_CCEL // 2, 0)
            pltpu.async_copy(buf_v.at[pl.ds(_HROWS, _HROWS)], half_out(ci, 1),
                             osemB)
        return carry0

    lax.fori_loop(0, _W, chunk_body, 0)
    pltpu.make_async_copy(buf_v.at[pl.ds(0, _HROWS)],
                          half_out(0, 0), osemA).wait()
    pltpu.make_async_copy(buf_v.at[pl.ds(_HROWS, _HROWS)],
                          half_out(0, 1), osemB).wait()


def _sc_call(tt, ints, flts, ttab, utab):
    mesh = plsc.VectorSubcoreMesh(core_axis_name="c", subcore_axis_name="s")
    return pl.kernel(
        _sc_body,
        out_type=jax.ShapeDtypeStruct((_M, _W, _H, _C, _B), jnp.float32),
        mesh=mesh,
        compiler_params=pltpu.CompilerParams(needs_layout_passes=False),
        scratch_types=[
            pltpu.VMEM((2, _CCEL, _B), jnp.int32),
            pltpu.VMEM((76, _B), jnp.int32),
            pltpu.VMEM((40, _B), jnp.float32),
            pltpu.VMEM((16,), jnp.float32),
            pltpu.VMEM((16, 16), jnp.float32),
            pltpu.VMEM((_CROWS, _B), jnp.float32),
            pltpu.SemaphoreType.DMA,
            pltpu.SemaphoreType.DMA,
            pltpu.SemaphoreType.DMA,
        ],
    )(tt, ints, flts, ttab, utab)


def kernel(position, units_mask, relic_positions, relic_mask, tile_type,
           normalized_reward, tile_table, unit_table):
    f32 = jnp.float32
    i32 = jnp.int32
    tt = tile_type.astype(i32).transpose(1, 2, 3, 0).reshape(_M, _CELLS, _B)
    x = position[..., 0].transpose(1, 2, 3, 0).reshape(_M, _T * _U, _B)
    y = position[..., 1].transpose(1, 2, 3, 0).reshape(_M, _T * _U, _B)
    rx = relic_positions[..., 0].transpose(1, 2, 0)
    ry = relic_positions[..., 1].transpose(1, 2, 0)
    ints = jnp.concatenate([x, y, rx, ry], axis=1)
    um = units_mask.astype(f32).transpose(1, 2, 3, 0).reshape(_M, _T * _U, _B)
    rm = relic_mask.astype(f32).transpose(1, 2, 0)
    nr = normalized_reward.astype(f32).T[:, None, :]
    flts = jnp.concatenate([um, rm, nr, jnp.zeros((_M, 1, _B), f32)], axis=1)
    ttab = jnp.concatenate([tile_table.reshape(12).astype(f32),
                            jnp.zeros((4,), f32)])
    utab = jnp.concatenate(
        [jnp.full((_U, 1), 1.0 / _U, f32), unit_table.astype(f32),
         jnp.zeros((_U, 7), f32)], axis=1)
    out = _sc_call(tt, ints, flts, ttab, utab)
    return out.transpose(4, 0, 1, 2, 3)
